# baseline (speedup 1.0000x reference)
import functools

import jax
import jax.numpy as jnp
from jax import lax
from jax.experimental import pallas as pl
from jax.experimental.pallas import tpu as pltpu
from jax.experimental.pallas import tpu_sc as plsc

NUM_CORES = 2
NUM_SUBCORES = 16
NW = NUM_CORES * NUM_SUBCORES
LANES = 16


def _emb_kernel(batch, seq_len, dim):
    s_per_w = seq_len // NW
    mesh = plsc.VectorSubcoreMesh(core_axis_name="c", subcore_axis_name="s")

    @functools.partial(
        pl.kernel,
        mesh=mesh,
        out_type=jax.ShapeDtypeStruct((batch, seq_len, dim), jnp.float32),
        scratch_types=[
            pltpu.VMEM((batch, s_per_w), jnp.int32),
            pltpu.VMEM((s_per_w, dim), jnp.float32),
            pltpu.VMEM((batch, s_per_w, dim), jnp.float32),
            pltpu.SemaphoreType.DMA,
            pltpu.SemaphoreType.DMA,
            pltpu.SemaphoreType.DMA((4,)),
            pltpu.SemaphoreType.DMA,
        ],
    )
    def emb(idx_hbm, table_hbm, pos_hbm, out_hbm, idx_v, pos_v, rows_v,
            sem_p, sem_i, sem_g, sem_o):
        wid = lax.axis_index("s") * NUM_CORES + lax.axis_index("c")
        base = wid * s_per_w
        cp_pos = pltpu.async_copy(pos_hbm.at[pl.ds(base, s_per_w)], pos_v,
                                  sem_p)
        idx_cps = [
            pltpu.async_copy(idx_hbm.at[b, pl.ds(base, s_per_w)], idx_v.at[b],
                             sem_i)
            for b in range(batch)
        ]
        for c in idx_cps:
            c.wait()
        gathers = [
            pltpu.async_copy(table_hbm.at[idx_v.at[b]], rows_v.at[b],
                             sem_g.at[b])
            for b in range(batch)
        ]
        cp_pos.wait()
        outs = []
        for b in range(batch):
            gathers[b].wait()

            def add_row(r, carry, b=b):
                for c in range(dim // LANES):
                    sl = pl.ds(c * LANES, LANES)
                    rows_v[b, r, sl] = rows_v[b, r, sl] + pos_v[r, sl]
                return carry

            lax.fori_loop(0, s_per_w, add_row, 0)
            outs.append(
                pltpu.async_copy(rows_v.at[b],
                                 out_hbm.at[b, pl.ds(base, s_per_w)], sem_o))
        for c in outs:
            c.wait()

    return emb


def kernel(inputs, W_word, P_pos):
    batch, seq_len = inputs.shape
    vocab, dim = W_word.shape
    return _emb_kernel(batch, seq_len, dim)(inputs, W_word, P_pos)
